# Initial kernel scaffold; baseline (speedup 1.0000x reference)
import jax
import jax.numpy as jnp
from jax.experimental import pallas as pl


def kernel(x, edge_index, edge_attr, batch, current_node_ids, action_mask, one_hot_goal, Wrel1, brel1, Wroot1, gn1_w, gn1_b, gn1_ms, Wrel2, brel2, Wroot2, gn2_w, gn2_b, gn2_ms, Wc, bc, Wvs, bvs, Wv, bv, Was, bas, Wa, ba):
    raise NotImplementedError("write your pallas kernel here")


# speedup vs baseline: 12.8845x; 12.8845x over previous
import functools

import jax
import jax.numpy as jnp
from jax import lax
from jax.experimental import pallas as pl
from jax.experimental.pallas import tpu as pltpu
from jax.experimental.pallas import tpu_sc as plsc

_NW = 32
_NSUB = 16
_CE = 128
_HP = 16


def _proj_body(x_ref, wp_ref, wr_ref, p_ref, r_ref):
    xv = x_ref[...]
    p_ref[...] = jnp.dot(xv, wp_ref[...], preferred_element_type=jnp.float32)
    r_ref[...] = jnp.dot(xv, wr_ref[...], preferred_element_type=jnp.float32)


def _proj(x, wp, wr):
    n = x.shape[0]
    return pl.pallas_call(
        _proj_body,
        out_shape=[
            jax.ShapeDtypeStruct((n, _HP), jnp.float32),
            jax.ShapeDtypeStruct((n, _HP), jnp.float32),
        ],
    )(x, wp, wr)


def _sc_edge_call(p, srcr, dstr, ewr, zeros):
    n = p.shape[0]
    nchunks = srcr.shape[1]
    rpt = n // _NSUB
    mesh = plsc.VectorSubcoreMesh(core_axis_name="c", subcore_axis_name="s")

    @functools.partial(
        pl.kernel,
        mesh=mesh,
        compiler_params=pltpu.CompilerParams(use_tc_tiling_on_sc=False),
        out_type=jax.ShapeDtypeStruct((2, n, _HP), jnp.float32),
        scratch_types=[
            pltpu.VMEM((nchunks, _CE), jnp.int32),
            pltpu.VMEM((nchunks, _CE), jnp.int32),
            pltpu.VMEM((_CE, _HP), jnp.float32),
            pltpu.VMEM((nchunks, _CE), jnp.float32),
            pltpu.VMEM_SHARED((n, _HP), jnp.float32),
            pltpu.SemaphoreType.DMA,
        ],
    )
    def sc_fn(p_hbm, src_hbm, dst_hbm, ew_hbm, z_hbm, out_hbm,
              srcv, dstv, rows, ewsm, aggsh, sem):
        c = lax.axis_index("c")
        s = lax.axis_index("s")
        t = c * _NSUB + s
        pltpu.sync_copy(z_hbm.at[pl.ds(s * rpt, rpt)],
                        aggsh.at[pl.ds(s * rpt, rpt)])
        pltpu.sync_copy(src_hbm.at[t], srcv)
        pltpu.sync_copy(dst_hbm.at[t], dstv)
        pltpu.sync_copy(ew_hbm.at[t], ewsm)
        plsc.subcore_barrier()

        def chunk_body(j, carry):
            pltpu.async_copy(p_hbm.at[srcv.at[j]], rows, sem).wait()

            def grp_body(g, cc):
                ew16 = ewsm[j, pl.ds(g * _HP, _HP)]
                for i in range(_HP):
                    bew = jnp.full((_HP,), ew16[i], dtype=jnp.float32)
                    rows[g * _HP + i, :] = rows[g * _HP + i, :] * bew
                return cc

            lax.fori_loop(0, _CE // _HP, grp_body, 0)
            pltpu.sync_copy(rows, aggsh.at[dstv.at[j]], add=True)
            return carry

        lax.fori_loop(0, nchunks, chunk_body, 0)
        plsc.subcore_barrier()
        pltpu.sync_copy(aggsh.at[pl.ds(s * rpt, rpt)],
                        out_hbm.at[c].at[pl.ds(s * rpt, rpt)])

    return sc_fn(p, srcr, dstr, ewr, zeros)


def _head_body(n, agg_ref, r_ref, b2d_ref, brow_ref, cni_ref, goal_ref,
               mask_ref, brel_ref, w_ref, b_ref, ms_ref, wasp_ref, basp_ref,
               wap_ref, ba_ref, wvsp_ref, bvsp_ref, wvp_ref, bv_ref, out_ref):
    f32 = jnp.float32
    nb = out_ref.shape[0]
    npad = r_ref.shape[0]
    y = agg_ref[0] + agg_ref[1] + r_ref[...] + brel_ref[...]

    onehot = (b2d_ref[...] ==
              lax.broadcasted_iota(jnp.int32, (npad, nb), 1)).astype(f32)
    onehot_t = (brow_ref[...] ==
                lax.broadcasted_iota(jnp.int32, (nb, npad), 0)).astype(f32)
    ones_bb = jnp.ones((nb, nb), f32)
    counts = jnp.dot(jnp.dot(onehot_t, onehot,
                             preferred_element_type=f32), ones_bb,
                     preferred_element_type=f32)
    seg_y = jnp.dot(onehot_t, y, preferred_element_type=f32)
    mean_ms = (seg_y / counts) * ms_ref[...]
    out = y - jnp.dot(onehot, mean_ms, preferred_element_type=f32)
    seg_o2 = jnp.dot(onehot_t, out * out, preferred_element_type=f32)
    std = jnp.sqrt(seg_o2 / counts + 1e-5)

    tril = (lax.broadcasted_iota(jnp.int32, (nb, nb), 0) >
            lax.broadcasted_iota(jnp.int32, (nb, nb), 1)).astype(f32)
    offsets = jnp.dot(tril, counts, preferred_element_type=f32)
    gidx = jnp.minimum(offsets[:, :1] + cni_ref[...], float(n - 1)
                       ).astype(jnp.int32)
    gsel = (lax.broadcasted_iota(jnp.int32, (nb, npad), 1) == gidx).astype(f32)
    rows_out = jnp.dot(gsel, out, preferred_element_type=f32)
    gb = jnp.dot(gsel, onehot, preferred_element_type=f32)
    std_sel = jnp.dot(gb, std, preferred_element_type=f32)
    h16 = jnp.maximum(w_ref[...] * rows_out / std_sel + b_ref[...], 0.0)

    xx = jnp.concatenate([h16, goal_ref[...]], axis=1)
    adv_h = jnp.maximum(
        jnp.dot(xx, wasp_ref[...], preferred_element_type=f32) + basp_ref[...],
        0.0)
    adv = jnp.dot(adv_h, wap_ref[...], preferred_element_type=f32) + ba_ref[...]
    val_h = jnp.maximum(
        jnp.dot(xx, wvsp_ref[...], preferred_element_type=f32) + bvsp_ref[...],
        0.0)
    val = jnp.dot(val_h, wvp_ref[...], preferred_element_type=f32) + bv_ref[...]
    mean_adv = jnp.mean(adv, axis=1, keepdims=True)
    q = val + adv - mean_adv
    out_ref[...] = jnp.where(mask_ref[...] == 0, -100000000.0, q)


def _head(n, nb, na, *args):
    return pl.pallas_call(
        functools.partial(_head_body, n),
        out_shape=jax.ShapeDtypeStruct((nb, na), jnp.float32),
    )(*args)


def kernel(x, edge_index, edge_attr, batch, current_node_ids, action_mask,
           one_hot_goal, Wrel1, brel1, Wroot1, gn1_w, gn1_b, gn1_ms, Wrel2,
           brel2, Wroot2, gn2_w, gn2_b, gn2_ms, Wc, bc, Wvs, bvs, Wv, bv,
           Was, bas, Wa, ba):
    f32 = jnp.float32
    n, d = x.shape
    e = edge_index.shape[1]
    nb = one_hot_goal.shape[0]
    na = action_mask.shape[1]
    h = Wrel1.shape[1]
    hh = Was.shape[1]
    goal = one_hot_goal.shape[1]

    npad = 128 * ((n + 127) // 128)
    xp = jnp.zeros((npad, d), f32).at[:n].set(x)
    wp = jnp.zeros((d, _HP), f32).at[:, :h].set(Wrel1)
    wr = jnp.zeros((d, _HP), f32).at[:, :h].set(Wroot1)
    p, r = _proj(xp, wp, wr)

    grain = _NW * _CE
    ep = grain * ((e + grain - 1) // grain)
    pad = ep - e
    fill = jnp.arange(pad, dtype=jnp.int32) % n
    srcr = jnp.concatenate([edge_index[0].astype(jnp.int32), fill]
                           ).reshape(_NW, -1, _CE)
    dstr = jnp.concatenate([edge_index[1].astype(jnp.int32), fill]
                           ).reshape(_NW, -1, _CE)
    ewr = jnp.concatenate([edge_attr, jnp.zeros((pad,), f32)]
                          ).reshape(_NW, -1, _CE)
    zeros = jnp.zeros((npad, _HP), f32)
    agg = _sc_edge_call(p, srcr, dstr, ewr, zeros)

    bpad = jnp.full((npad,), nb, jnp.int32).at[:n].set(batch.astype(jnp.int32))
    batch2d = bpad.reshape(npad, 1)
    batchrow = bpad.reshape(1, npad)
    cni = current_node_ids.astype(f32).reshape(nb, 1)
    brel_row = jnp.zeros((1, _HP), f32).at[0, :h].set(brel1)
    w_row = jnp.zeros((1, _HP), f32).at[0, :h].set(gn1_w)
    b_row = jnp.zeros((1, _HP), f32).at[0, :h].set(gn1_b)
    ms_row = jnp.zeros((1, _HP), f32).at[0, :h].set(gn1_ms)
    wasp = (jnp.zeros((2 * _HP, _HP), f32)
            .at[:h, :hh].set(Was[:h])
            .at[_HP:_HP + goal, :hh].set(Was[h:]))
    basp = jnp.zeros((1, _HP), f32).at[0, :hh].set(bas)
    wap = jnp.zeros((_HP, na), f32).at[:hh].set(Wa)
    ba_row = ba.reshape(1, na)
    wvsp = (jnp.zeros((2 * _HP, _HP), f32)
            .at[:h, :hh].set(Wvs[:h])
            .at[_HP:_HP + goal, :hh].set(Wvs[h:]))
    bvsp = jnp.zeros((1, _HP), f32).at[0, :hh].set(bvs)
    wvp = jnp.zeros((_HP, na), f32).at[:hh].set(jnp.tile(Wv, (1, na)))
    bv11 = bv.reshape(1, 1)

    return _head(n, nb, na, agg, r, batch2d, batchrow, cni, one_hot_goal,
                 action_mask, brel_row, w_row, b_row, ms_row, wasp, basp,
                 wap, ba_row, wvsp, bvsp, wvp, bv11)


# speedup vs baseline: 17.4831x; 1.3569x over previous
import functools

import jax
import jax.numpy as jnp
from jax import lax
from jax.experimental import pallas as pl
from jax.experimental.pallas import tpu as pltpu
from jax.experimental.pallas import tpu_sc as plsc

_NW = 32
_NSUB = 16
_CE = 128
_HP = 16


def _proj_body(x_ref, wp_ref, wr_ref, p_ref, r_ref):
    xv = x_ref[...]
    p_ref[...] = jnp.dot(xv, wp_ref[...], preferred_element_type=jnp.float32)
    r_ref[...] = jnp.dot(xv, wr_ref[...], preferred_element_type=jnp.float32)


def _proj(x, wp, wr):
    n = x.shape[0]
    return pl.pallas_call(
        _proj_body,
        out_shape=[
            jax.ShapeDtypeStruct((n, _HP), jnp.float32),
            jax.ShapeDtypeStruct((n, _HP), jnp.float32),
        ],
    )(x, wp, wr)


def _sc_edge_call(p, srcr, dstr, ewr, zeros):
    n = p.shape[0]
    nchunks = srcr.shape[1]
    rpt = n // _NSUB
    mesh = plsc.VectorSubcoreMesh(core_axis_name="c", subcore_axis_name="s")

    @functools.partial(
        pl.kernel,
        mesh=mesh,
        compiler_params=pltpu.CompilerParams(use_tc_tiling_on_sc=False),
        out_type=jax.ShapeDtypeStruct((2, n, _HP), jnp.float32),
        scratch_types=[
            pltpu.VMEM((nchunks, _CE), jnp.int32),
            pltpu.VMEM((nchunks, _CE), jnp.int32),
            pltpu.VMEM((nchunks, _CE), jnp.float32),
            pltpu.VMEM((2, _CE, _HP), jnp.float32),
            pltpu.VMEM((2, _CE, _HP), jnp.float32),
            pltpu.VMEM_SHARED((n, _HP), jnp.float32),
            pltpu.SemaphoreType.DMA,
            pltpu.SemaphoreType.DMA,
            pltpu.SemaphoreType.DMA,
            pltpu.SemaphoreType.DMA,
        ],
    )
    def sc_fn(p_hbm, src_hbm, dst_hbm, ew_hbm, z_hbm, out_hbm,
              srcv, dstv, ewv, grows, srows, aggsh, sg0, sg1, ss0, ss1):
        c = lax.axis_index("c")
        s = lax.axis_index("s")
        t = c * _NSUB + s
        sg = (sg0, sg1)
        ss = (ss0, ss1)
        pltpu.sync_copy(z_hbm.at[pl.ds(s * rpt, rpt)],
                        aggsh.at[pl.ds(s * rpt, rpt)])
        pltpu.sync_copy(src_hbm.at[t], srcv)
        pltpu.sync_copy(dst_hbm.at[t], dstv)
        pltpu.sync_copy(ew_hbm.at[t], ewv)
        plsc.subcore_barrier()

        def issue_gather(j, b):
            pltpu.async_copy(p_hbm.at[srcv.at[j]], grows.at[b], sg[b])

        def wait_gather(j, b):
            pltpu.make_async_copy(p_hbm.at[srcv.at[j]], grows.at[b],
                                  sg[b]).wait()

        def issue_scatter(j, b):
            pltpu.async_copy(srows.at[b], aggsh.at[dstv.at[j]], ss[b],
                             add=True)

        def wait_scatter(j, b):
            pltpu.make_async_copy(srows.at[b], aggsh.at[dstv.at[j]],
                                  ss[b]).wait()

        def mult(j, b):
            def grp_body(g, cc):
                ew16 = ewv[j, pl.ds(g * _HP, _HP)]
                for i in range(_HP):
                    bew = jnp.full((_HP,), ew16[i], dtype=jnp.float32)
                    srows[b, g * _HP + i, :] = grows[b, g * _HP + i, :] * bew
                return cc

            lax.fori_loop(0, _CE // _HP, grp_body, 0)

        issue_gather(0, 0)
        issue_gather(1, 1)
        wait_gather(0, 0)
        mult(0, 0)
        issue_scatter(0, 0)
        issue_gather(2, 0)
        wait_gather(1, 1)
        mult(1, 1)
        issue_scatter(1, 1)

        def pair_body(jj, carry):
            j0 = 2 + jj * 2
            for b in (0, 1):
                jb = j0 + b

                @pl.when(jb + 1 < nchunks)
                def _():
                    issue_gather(jb + 1, 1 - b)

                wait_gather(jb, b)
                wait_scatter(jb - 2, b)
                mult(jb, b)
                issue_scatter(jb, b)
            return carry

        lax.fori_loop(0, (nchunks - 2) // 2, pair_body, 0)
        wait_scatter(nchunks - 2, 0)
        wait_scatter(nchunks - 1, 1)
        plsc.subcore_barrier()
        pltpu.sync_copy(aggsh.at[pl.ds(s * rpt, rpt)],
                        out_hbm.at[c].at[pl.ds(s * rpt, rpt)])

    return sc_fn(p, srcr, dstr, ewr, zeros)


def _head_body(n, agg_ref, r_ref, b2d_ref, brow_ref, cni_ref, goal_ref,
               mask_ref, brel_ref, w_ref, b_ref, ms_ref, wasp_ref, basp_ref,
               wap_ref, ba_ref, wvsp_ref, bvsp_ref, wvp_ref, bv_ref, out_ref):
    f32 = jnp.float32
    nb = out_ref.shape[0]
    npad = r_ref.shape[0]
    y = agg_ref[0] + agg_ref[1] + r_ref[...] + brel_ref[...]

    onehot = (b2d_ref[...] ==
              lax.broadcasted_iota(jnp.int32, (npad, nb), 1)).astype(f32)
    onehot_t = (brow_ref[...] ==
                lax.broadcasted_iota(jnp.int32, (nb, npad), 0)).astype(f32)
    ones_bb = jnp.ones((nb, nb), f32)
    counts = jnp.dot(jnp.dot(onehot_t, onehot,
                             preferred_element_type=f32), ones_bb,
                     preferred_element_type=f32)
    seg_y = jnp.dot(onehot_t, y, preferred_element_type=f32)
    mean_ms = (seg_y / counts) * ms_ref[...]
    out = y - jnp.dot(onehot, mean_ms, preferred_element_type=f32)
    seg_o2 = jnp.dot(onehot_t, out * out, preferred_element_type=f32)
    std = jnp.sqrt(seg_o2 / counts + 1e-5)

    tril = (lax.broadcasted_iota(jnp.int32, (nb, nb), 0) >
            lax.broadcasted_iota(jnp.int32, (nb, nb), 1)).astype(f32)
    offsets = jnp.dot(tril, counts, preferred_element_type=f32)
    gidx = jnp.minimum(offsets[:, :1] + cni_ref[...], float(n - 1)
                       ).astype(jnp.int32)
    gsel = (lax.broadcasted_iota(jnp.int32, (nb, npad), 1) == gidx).astype(f32)
    rows_out = jnp.dot(gsel, out, preferred_element_type=f32)
    gb = jnp.dot(gsel, onehot, preferred_element_type=f32)
    std_sel = jnp.dot(gb, std, preferred_element_type=f32)
    h16 = jnp.maximum(w_ref[...] * rows_out / std_sel + b_ref[...], 0.0)

    xx = jnp.concatenate([h16, goal_ref[...]], axis=1)
    adv_h = jnp.maximum(
        jnp.dot(xx, wasp_ref[...], preferred_element_type=f32) + basp_ref[...],
        0.0)
    adv = jnp.dot(adv_h, wap_ref[...], preferred_element_type=f32) + ba_ref[...]
    val_h = jnp.maximum(
        jnp.dot(xx, wvsp_ref[...], preferred_element_type=f32) + bvsp_ref[...],
        0.0)
    val = jnp.dot(val_h, wvp_ref[...], preferred_element_type=f32) + bv_ref[...]
    mean_adv = jnp.mean(adv, axis=1, keepdims=True)
    q = val + adv - mean_adv
    out_ref[...] = jnp.where(mask_ref[...] == 0, -100000000.0, q)


def _head(n, nb, na, *args):
    return pl.pallas_call(
        functools.partial(_head_body, n),
        out_shape=jax.ShapeDtypeStruct((nb, na), jnp.float32),
    )(*args)


def kernel(x, edge_index, edge_attr, batch, current_node_ids, action_mask,
           one_hot_goal, Wrel1, brel1, Wroot1, gn1_w, gn1_b, gn1_ms, Wrel2,
           brel2, Wroot2, gn2_w, gn2_b, gn2_ms, Wc, bc, Wvs, bvs, Wv, bv,
           Was, bas, Wa, ba):
    f32 = jnp.float32
    n, d = x.shape
    e = edge_index.shape[1]
    nb = one_hot_goal.shape[0]
    na = action_mask.shape[1]
    h = Wrel1.shape[1]
    hh = Was.shape[1]
    goal = one_hot_goal.shape[1]

    npad = 128 * ((n + 127) // 128)
    xp = jnp.zeros((npad, d), f32).at[:n].set(x)
    wp = jnp.zeros((d, _HP), f32).at[:, :h].set(Wrel1)
    wr = jnp.zeros((d, _HP), f32).at[:, :h].set(Wroot1)
    p, r = _proj(xp, wp, wr)

    grain = _NW * _CE * 2
    ep = grain * ((e + grain - 1) // grain)
    pad = ep - e
    fill = jnp.arange(pad, dtype=jnp.int32) % n
    srcr = jnp.concatenate([edge_index[0].astype(jnp.int32), fill]
                           ).reshape(_NW, -1, _CE)
    dstr = jnp.concatenate([edge_index[1].astype(jnp.int32), fill]
                           ).reshape(_NW, -1, _CE)
    ewr = jnp.concatenate([edge_attr, jnp.zeros((pad,), f32)]
                          ).reshape(_NW, -1, _CE)
    zeros = jnp.zeros((npad, _HP), f32)
    agg = _sc_edge_call(p, srcr, dstr, ewr, zeros)

    bpad = jnp.full((npad,), nb, jnp.int32).at[:n].set(batch.astype(jnp.int32))
    batch2d = bpad.reshape(npad, 1)
    batchrow = bpad.reshape(1, npad)
    cni = current_node_ids.astype(f32).reshape(nb, 1)
    brel_row = jnp.zeros((1, _HP), f32).at[0, :h].set(brel1)
    w_row = jnp.zeros((1, _HP), f32).at[0, :h].set(gn1_w)
    b_row = jnp.zeros((1, _HP), f32).at[0, :h].set(gn1_b)
    ms_row = jnp.zeros((1, _HP), f32).at[0, :h].set(gn1_ms)
    wasp = (jnp.zeros((2 * _HP, _HP), f32)
            .at[:h, :hh].set(Was[:h])
            .at[_HP:_HP + goal, :hh].set(Was[h:]))
    basp = jnp.zeros((1, _HP), f32).at[0, :hh].set(bas)
    wap = jnp.zeros((_HP, na), f32).at[:hh].set(Wa)
    ba_row = ba.reshape(1, na)
    wvsp = (jnp.zeros((2 * _HP, _HP), f32)
            .at[:h, :hh].set(Wvs[:h])
            .at[_HP:_HP + goal, :hh].set(Wvs[h:]))
    bvsp = jnp.zeros((1, _HP), f32).at[0, :hh].set(bvs)
    wvp = jnp.zeros((_HP, na), f32).at[:hh].set(jnp.tile(Wv, (1, na)))
    bv11 = bv.reshape(1, 1)

    return _head(n, nb, na, agg, r, batch2d, batchrow, cni, one_hot_goal,
                 action_mask, brel_row, w_row, b_row, ms_row, wasp, basp,
                 wap, ba_row, wvsp, bvsp, wvp, bv11)


# speedup vs baseline: 19.7887x; 1.1319x over previous
import functools

import jax
import jax.numpy as jnp
from jax import lax
from jax.experimental import pallas as pl
from jax.experimental.pallas import tpu as pltpu
from jax.experimental.pallas import tpu_sc as plsc

_NW = 32
_NSUB = 16
_CE = 128
_HP = 16


def _row16(vec):
    row = vec.reshape(1, -1).astype(jnp.float32)
    return jnp.concatenate(
        [row, jnp.zeros((1, _HP - row.shape[1]), jnp.float32)], axis=1)


def _proj_body(npad, x_ref, wp_ref, wr_ref, p_ref, r_ref):
    n = x_ref.shape[0]
    xv = x_ref[...]
    zcol = jnp.zeros((n, _HP - wp_ref.shape[1]), jnp.float32)
    zrow = jnp.zeros((npad - n, _HP), jnp.float32)
    p_ref[...] = jnp.concatenate([
        jnp.concatenate(
            [jnp.dot(xv, wp_ref[...], preferred_element_type=jnp.float32),
             zcol], axis=1), zrow], axis=0)
    r_ref[...] = jnp.concatenate([
        jnp.concatenate(
            [jnp.dot(xv, wr_ref[...], preferred_element_type=jnp.float32),
             zcol], axis=1), zrow], axis=0)


def _proj(npad, x, wp, wr):
    return pl.pallas_call(
        functools.partial(_proj_body, npad),
        out_shape=[
            jax.ShapeDtypeStruct((npad, _HP), jnp.float32),
            jax.ShapeDtypeStruct((npad, _HP), jnp.float32),
        ],
    )(x, wp, wr)


def _sc_edge_call(p, srcr, dstr, ewr):
    n = p.shape[0]
    nchunks = srcr.shape[1]
    rpt = n // _NSUB
    mesh = plsc.VectorSubcoreMesh(core_axis_name="c", subcore_axis_name="s")

    @functools.partial(
        pl.kernel,
        mesh=mesh,
        compiler_params=pltpu.CompilerParams(use_tc_tiling_on_sc=False),
        out_type=jax.ShapeDtypeStruct((2, n, _HP), jnp.float32),
        scratch_types=[
            pltpu.VMEM((nchunks, _CE), jnp.int32),
            pltpu.VMEM((nchunks, _CE), jnp.int32),
            pltpu.VMEM((nchunks, _CE), jnp.float32),
            pltpu.VMEM((2, _CE, _HP), jnp.float32),
            pltpu.VMEM((2, _CE, _HP), jnp.float32),
            pltpu.VMEM_SHARED((n, _HP), jnp.float32),
            pltpu.SemaphoreType.DMA,
            pltpu.SemaphoreType.DMA,
            pltpu.SemaphoreType.DMA,
            pltpu.SemaphoreType.DMA,
        ],
    )
    def sc_fn(p_hbm, src_hbm, dst_hbm, ew_hbm, out_hbm,
              srcv, dstv, ewv, grows, srows, aggsh, sg0, sg1, ss0, ss1):
        c = lax.axis_index("c")
        s = lax.axis_index("s")
        t = c * _NSUB + s
        sg = (sg0, sg1)
        ss = (ss0, ss1)

        def zrow_body(i, cc):
            srows[0, i, :] = jnp.zeros((_HP,), jnp.float32)
            return cc

        lax.fori_loop(0, _CE, zrow_body, 0)
        nfull = rpt // _CE
        for k in range(nfull):
            pltpu.sync_copy(srows.at[0],
                            aggsh.at[pl.ds(s * rpt + k * _CE, _CE)])
        rem = rpt - nfull * _CE
        if rem:
            pltpu.sync_copy(srows.at[0, pl.ds(0, rem)],
                            aggsh.at[pl.ds(s * rpt + nfull * _CE, rem)])
        pltpu.sync_copy(src_hbm.at[t], srcv)
        pltpu.sync_copy(dst_hbm.at[t], dstv)
        pltpu.sync_copy(ew_hbm.at[t], ewv)
        plsc.subcore_barrier()

        def issue_gather(j, b):
            pltpu.async_copy(p_hbm.at[srcv.at[j]], grows.at[b], sg[b])

        def wait_gather(j, b):
            pltpu.make_async_copy(p_hbm.at[srcv.at[j]], grows.at[b],
                                  sg[b]).wait()

        def issue_scatter(j, b):
            pltpu.async_copy(srows.at[b], aggsh.at[dstv.at[j]], ss[b],
                             add=True)

        def wait_scatter(j, b):
            pltpu.make_async_copy(srows.at[b], aggsh.at[dstv.at[j]],
                                  ss[b]).wait()

        def mult(j, b):
            def grp_body(g, cc):
                ew16 = ewv[j, pl.ds(g * _HP, _HP)]
                for i in range(_HP):
                    bew = jnp.full((_HP,), ew16[i], dtype=jnp.float32)
                    srows[b, g * _HP + i, :] = grows[b, g * _HP + i, :] * bew
                return cc

            lax.fori_loop(0, _CE // _HP, grp_body, 0)

        issue_gather(0, 0)
        issue_gather(1, 1)
        wait_gather(0, 0)
        mult(0, 0)
        issue_scatter(0, 0)
        issue_gather(2, 0)
        wait_gather(1, 1)
        mult(1, 1)
        issue_scatter(1, 1)

        def pair_body(jj, carry):
            j0 = 2 + jj * 2
            for b in (0, 1):
                jb = j0 + b

                @pl.when(jb + 1 < nchunks)
                def _():
                    issue_gather(jb + 1, 1 - b)

                wait_gather(jb, b)
                wait_scatter(jb - 2, b)
                mult(jb, b)
                issue_scatter(jb, b)
            return carry

        lax.fori_loop(0, (nchunks - 2) // 2, pair_body, 0)
        wait_scatter(nchunks - 2, 0)
        wait_scatter(nchunks - 1, 1)
        plsc.subcore_barrier()
        pltpu.sync_copy(aggsh.at[pl.ds(s * rpt, rpt)],
                        out_hbm.at[c].at[pl.ds(s * rpt, rpt)])

    return sc_fn(p, srcr, dstr, ewr)


def _head_body(n, h, agg_ref, r_ref, batch_ref, cni_ref, goal_ref, mask_ref,
               brel_ref, gnw_ref, gnb_ref, gnms_ref, was_ref, bas_ref,
               wa_ref, ba_ref, wvs_ref, bvs_ref, wv_ref, bv_ref, out_ref):
    f32 = jnp.float32
    nb = out_ref.shape[0]
    y = (agg_ref[0] + agg_ref[1] + r_ref[...] + _row16(brel_ref[...]))[:n]
    brow = batch_ref[...].reshape(1, n)

    onehot_t = (brow ==
                lax.broadcasted_iota(jnp.int32, (nb, n), 0)).astype(f32)
    ii = lax.broadcasted_iota(jnp.int32, (nb, nb), 0)
    jj = lax.broadcasted_iota(jnp.int32, (nb, nb), 1)
    eye = (ii == jj).astype(f32)
    tril = (ii > jj).astype(f32)
    ones_bb = jnp.ones((nb, nb), f32)
    counts_col = jnp.sum(onehot_t, axis=1, keepdims=True)
    counts = jnp.dot(eye * counts_col, ones_bb, preferred_element_type=f32)
    seg_y = jnp.dot(onehot_t, y, preferred_element_type=f32)
    mean_ms = (seg_y / counts) * _row16(gnms_ref[...])
    seg_y2 = jnp.dot(onehot_t, y * y, preferred_element_type=f32)
    var = (seg_y2 - 2.0 * mean_ms * seg_y
           + counts * mean_ms * mean_ms) / counts
    std = jnp.sqrt(var + 1e-5)

    offsets = jnp.dot(tril, counts, preferred_element_type=f32)
    cni_row = cni_ref[...].reshape(1, nb).astype(f32)
    cni_col = jnp.dot(eye * cni_row, ones_bb, preferred_element_type=f32)
    gidx_full = jnp.minimum(offsets + cni_col, float(n - 1))
    gidx_col = gidx_full[:, :1].astype(jnp.int32)
    gsel = (lax.broadcasted_iota(jnp.int32, (nb, n), 1) == gidx_col
            ).astype(f32)
    y_sel = jnp.dot(gsel, y, preferred_element_type=f32)
    ends_row = jnp.dot(ones_bb, eye * (offsets + counts),
                       preferred_element_type=f32)
    bsel_col = jnp.sum((gidx_full >= ends_row).astype(f32), axis=1,
                       keepdims=True)
    gb = (lax.broadcasted_iota(jnp.int32, (nb, nb), 1).astype(f32) ==
          bsel_col).astype(f32)
    std_sel = jnp.dot(gb, std, preferred_element_type=f32)
    m_sel = jnp.dot(gb, mean_ms, preferred_element_type=f32)
    h16 = jnp.maximum(
        _row16(gnw_ref[...]) * (y_sel - m_sel) / std_sel
        + _row16(gnb_ref[...]), 0.0)

    xx = jnp.concatenate([h16[:, :h], goal_ref[...]], axis=1)
    adv_h = jnp.maximum(
        jnp.dot(xx, was_ref[...], preferred_element_type=f32)
        + bas_ref[...].reshape(1, -1), 0.0)
    adv = (jnp.dot(adv_h, wa_ref[...], preferred_element_type=f32)
           + ba_ref[...].reshape(1, -1))
    val_h = jnp.maximum(
        jnp.dot(xx, wvs_ref[...], preferred_element_type=f32)
        + bvs_ref[...].reshape(1, -1), 0.0)
    val = jnp.sum(val_h * wv_ref[...], axis=1, keepdims=True)
    mean_adv = jnp.mean(adv, axis=1, keepdims=True)
    q = val + adv - mean_adv + bv_ref[...].reshape(1, 1)
    out_ref[...] = jnp.where(mask_ref[...] == 0, -100000000.0, q)


def _head(n, h, nb, na, *args):
    return pl.pallas_call(
        functools.partial(_head_body, n, h),
        out_shape=jax.ShapeDtypeStruct((nb, na), jnp.float32),
    )(*args)


def kernel(x, edge_index, edge_attr, batch, current_node_ids, action_mask,
           one_hot_goal, Wrel1, brel1, Wroot1, gn1_w, gn1_b, gn1_ms, Wrel2,
           brel2, Wroot2, gn2_w, gn2_b, gn2_ms, Wc, bc, Wvs, bvs, Wv, bv,
           Was, bas, Wa, ba):
    f32 = jnp.float32
    n, d = x.shape
    e = edge_index.shape[1]
    nb = one_hot_goal.shape[0]
    na = action_mask.shape[1]
    h = Wrel1.shape[1]

    npad = 128 * ((n + 127) // 128)
    p, r = _proj(npad, x, Wrel1, Wroot1)

    grain = _NW * _CE * 2
    ep = grain * ((e + grain - 1) // grain)
    pad = ep - e
    fill = jnp.arange(pad, dtype=jnp.int32) % n
    srcr = jnp.concatenate([edge_index[0].astype(jnp.int32), fill]
                           ).reshape(_NW, -1, _CE)
    dstr = jnp.concatenate([edge_index[1].astype(jnp.int32), fill]
                           ).reshape(_NW, -1, _CE)
    ewr = jnp.concatenate([edge_attr, jnp.zeros((pad,), f32)]
                          ).reshape(_NW, -1, _CE)
    agg = _sc_edge_call(p, srcr, dstr, ewr)

    return _head(n, h, nb, na, agg, r, batch.astype(jnp.int32),
                 current_node_ids.astype(jnp.int32), one_hot_goal,
                 action_mask, brel1, gn1_w, gn1_b, gn1_ms, Was, bas, Wa, ba,
                 Wvs, bvs, Wv.reshape(1, -1), bv)
